# baseline (speedup 1.0000x reference)
import jax
import jax.numpy as jnp
from jax.experimental import pallas as pl
from jax.experimental.pallas import tpu as pltpu

_NBUF = (3, 1)
_CP = 1601
_CP_LAST = max(1601 - (1601 // _CP) * _CP, 1)


def _schedule(NSEG, P):
    chunks = []
    counts = [0, 0]
    for seg in range(NSEG):
        r = 0
        while r < P:
            n = min(_CP, P - r)
            pool = 0 if n == _CP else 1
            idx = counts[pool]
            counts[pool] += 1
            chunks.append((seg, r, n, pool, idx % _NBUF[pool], idx))
            r += n
    return chunks


def _body(ids_ref, hid_ref, emb_ref, gate_ref, out_ref,
          rows_ref, in0, out0, in1, out1,
          in_sems0, out_sems0, in_sems1, out_sems1):
    NSEG, P, H = hid_ref.shape
    T = emb_ref.shape[1]
    chunks = _schedule(NSEG, P)
    total = len(chunks)
    inbufs = (in0, in1)
    outbufs = (out0, out1)
    in_sems = (in_sems0, in_sems1)
    out_sems = (out_sems0, out_sems1)

    g = jnp.tanh(gate_ref[...])
    for seg in range(NSEG):
        b, t = divmod(seg, T)
        row = ids_ref[b]
        rows_ref[seg] = emb_ref[row, t] * g

    def in_copy(k):
        seg, row0, nrows, pool, slot, _ = chunks[k]
        return pltpu.make_async_copy(
            hid_ref.at[seg, pl.ds(row0, nrows)],
            inbufs[pool].at[slot],
            in_sems[pool].at[slot],
        )

    def out_copy(k):
        seg, row0, nrows, pool, slot, _ = chunks[k]
        return pltpu.make_async_copy(
            outbufs[pool].at[slot],
            out_ref.at[seg, pl.ds(row0, nrows)],
            out_sems[pool].at[slot],
        )

    by_pool_idx = {(c[3], c[5]): i for i, c in enumerate(chunks)}
    next_start = [by_pool_idx.get((c[3], c[5] + _NBUF[c[3]])) for c in chunks]
    prev_out = [by_pool_idx.get((c[3], c[5] - _NBUF[c[3]])) for c in chunks]

    for k in range(total):
        if chunks[k][5] < _NBUF[chunks[k][3]]:
            in_copy(k).start()

    for k in range(total):
        seg, row0, nrows, pool, slot, _ = chunks[k]
        in_copy(k).wait()
        if prev_out[k] is not None:
            out_copy(prev_out[k]).wait()
        outbufs[pool][slot] = inbufs[pool][slot] + rows_ref[seg]
        out_copy(k).start()
        if next_start[k] is not None:
            in_copy(next_start[k]).start()

    for k in range(total):
        if next_start[k] is None:
            out_copy(k).wait()


def kernel(hidden_state, aspect_ratio_ids, embedding_table, gate):
    B, T, P, H = hidden_state.shape
    hid = hidden_state.reshape(B * T, P, H)
    emb = embedding_table.reshape(-1, T, 1, H)
    ids = aspect_ratio_ids.astype(jnp.int32)
    gate2d = gate.reshape(1, 1)

    grid_spec = pltpu.PrefetchScalarGridSpec(
        num_scalar_prefetch=1,
        grid=(1,),
        in_specs=[
            pl.BlockSpec(memory_space=pl.ANY),
            pl.BlockSpec((emb.shape[0], T, 1, H), lambda i, ids_ref: (0, 0, 0, 0)),
            pl.BlockSpec((1, 1), lambda i, ids_ref: (0, 0)),
        ],
        out_specs=pl.BlockSpec(memory_space=pl.ANY),
        scratch_shapes=[
            pltpu.VMEM((B * T, 1, H), jnp.float32),
            pltpu.VMEM((_NBUF[0], _CP, H), jnp.float32),
            pltpu.VMEM((_NBUF[0], _CP, H), jnp.float32),
            pltpu.VMEM((_NBUF[1], _CP_LAST, H), jnp.float32),
            pltpu.VMEM((_NBUF[1], _CP_LAST, H), jnp.float32),
            pltpu.SemaphoreType.DMA((_NBUF[0],)),
            pltpu.SemaphoreType.DMA((_NBUF[0],)),
            pltpu.SemaphoreType.DMA((_NBUF[1],)),
            pltpu.SemaphoreType.DMA((_NBUF[1],)),
        ],
    )
    out = pl.pallas_call(
        _body,
        grid_spec=grid_spec,
        out_shape=jax.ShapeDtypeStruct((B * T, P, H), hidden_state.dtype),
    )(ids, hid, emb, gate2d)
    return out.reshape(B, T, P, H)


# speedup vs baseline: 3.6363x; 3.6363x over previous
import jax
import jax.numpy as jnp
from jax.experimental import pallas as pl
from jax.experimental.pallas import tpu as pltpu

_NBUF = (8, 4)
_CP = 232
_CP_LAST = 1601 - (1601 // _CP) * _CP


def _schedule(B, T, P):
    chunks = []
    counts = [0, 0]
    for b in range(B):
        for t in range(T):
            r = 0
            while r < P:
                n = min(_CP, P - r)
                pool = 0 if n == _CP else 1
                idx = counts[pool]
                counts[pool] += 1
                chunks.append((b, t, r, n, pool, idx % _NBUF[pool], idx))
                r += n
    return chunks


def _body(ids_ref, hid_ref, emb_ref, gate_ref, out_ref,
          rows_ref, in0, out0, in1, out1,
          in_sems0, out_sems0, in_sems1, out_sems1):
    B, T, P, H = hid_ref.shape
    chunks = _schedule(B, T, P)
    inbufs = (in0, in1)
    outbufs = (out0, out1)
    in_sems = (in_sems0, in_sems1)
    out_sems = (out_sems0, out_sems1)

    g = jnp.tanh(gate_ref[...])
    for seg in range(B * T):
        b, t = divmod(seg, T)
        row = ids_ref[b]
        rows_ref[seg] = emb_ref[row, t] * g

    def in_copy(k):
        b, t, row0, nrows, pool, slot, _ = chunks[k]
        return pltpu.make_async_copy(
            hid_ref.at[b, t, pl.ds(row0, nrows)],
            inbufs[pool].at[slot],
            in_sems[pool].at[slot],
        )

    def out_copy(k):
        b, t, row0, nrows, pool, slot, _ = chunks[k]
        return pltpu.make_async_copy(
            outbufs[pool].at[slot],
            out_ref.at[b, t, pl.ds(row0, nrows)],
            out_sems[pool].at[slot],
        )

    by_pool_idx = {(c[4], c[6]): i for i, c in enumerate(chunks)}
    next_start = [by_pool_idx.get((c[4], c[6] + _NBUF[c[4]])) for c in chunks]
    prev_out = [by_pool_idx.get((c[4], c[6] - _NBUF[c[4]])) for c in chunks]

    for k in range(len(chunks)):
        if chunks[k][6] < _NBUF[chunks[k][4]]:
            in_copy(k).start()

    for k in range(len(chunks)):
        b, t, row0, nrows, pool, slot, _ = chunks[k]
        in_copy(k).wait()
        if prev_out[k] is not None:
            out_copy(prev_out[k]).wait()
        outbufs[pool][slot] = inbufs[pool][slot] + rows_ref[b * T + t]
        out_copy(k).start()
        if next_start[k] is not None:
            in_copy(next_start[k]).start()

    for k in range(len(chunks)):
        if next_start[k] is None:
            out_copy(k).wait()


def kernel(hidden_state, aspect_ratio_ids, embedding_table, gate):
    B, T, P, H = hidden_state.shape
    emb = embedding_table.reshape(-1, T, 1, H)
    ids = aspect_ratio_ids.astype(jnp.int32)
    gate2d = gate.reshape(1, 1)

    grid_spec = pltpu.PrefetchScalarGridSpec(
        num_scalar_prefetch=1,
        grid=(1,),
        in_specs=[
            pl.BlockSpec(memory_space=pl.ANY),
            pl.BlockSpec((emb.shape[0], T, 1, H), lambda i, ids_ref: (0, 0, 0, 0)),
            pl.BlockSpec((1, 1), lambda i, ids_ref: (0, 0)),
        ],
        out_specs=pl.BlockSpec(memory_space=pl.ANY),
        scratch_shapes=[
            pltpu.VMEM((B * T, 1, H), jnp.float32),
            pltpu.VMEM((_NBUF[0], _CP, H), jnp.float32),
            pltpu.VMEM((_NBUF[0], _CP, H), jnp.float32),
            pltpu.VMEM((_NBUF[1], _CP_LAST, H), jnp.float32),
            pltpu.VMEM((_NBUF[1], _CP_LAST, H), jnp.float32),
            pltpu.SemaphoreType.DMA((_NBUF[0],)),
            pltpu.SemaphoreType.DMA((_NBUF[0],)),
            pltpu.SemaphoreType.DMA((_NBUF[1],)),
            pltpu.SemaphoreType.DMA((_NBUF[1],)),
        ],
    )
    return pl.pallas_call(
        _body,
        grid_spec=grid_spec,
        out_shape=jax.ShapeDtypeStruct((B, T, P, H), hidden_state.dtype),
    )(ids, hidden_state, emb, gate2d)


# speedup vs baseline: 3.6374x; 1.0003x over previous
import jax
import jax.numpy as jnp
from jax.experimental import pallas as pl
from jax.experimental.pallas import tpu as pltpu

_NBUF = (8, 4)
_CP = 232
_CP_LAST = 1601 - (1601 // _CP) * _CP


def _schedule(B, T, P):
    chunks = []
    counts = [0, 0]
    for b in range(B):
        for t in range(T):
            r = 0
            while r < P:
                n = min(_CP, P - r)
                pool = 0 if n == _CP else 1
                idx = counts[pool]
                counts[pool] += 1
                chunks.append((b, t, r, n, pool, idx % _NBUF[pool], idx))
                r += n
    return chunks


def _body(ids_ref, hid_ref, emb_ref, gate_ref, out_ref,
          rows_ref, in0, out0, in1, out1,
          in_sems0, out_sems0, in_sems1, out_sems1):
    B, T, P, H = hid_ref.shape
    chunks = _schedule(B, T, P)
    inbufs = (in0, in1)
    outbufs = (out0, out1)
    in_sems = (in_sems0, in_sems1)
    out_sems = (out_sems0, out_sems1)

    g = jnp.tanh(gate_ref[...])
    for seg in range(B * T):
        b, t = divmod(seg, T)
        row = ids_ref[b]
        rows_ref[seg] = emb_ref[row, t] * g

    def in_copy(k):
        b, t, row0, nrows, pool, slot, _ = chunks[k]
        return pltpu.make_async_copy(
            hid_ref.at[b, t, pl.ds(row0, nrows)],
            inbufs[pool].at[slot],
            in_sems[pool].at[slot],
        )

    def out_copy(k):
        b, t, row0, nrows, pool, slot, _ = chunks[k]
        return pltpu.make_async_copy(
            outbufs[pool].at[slot],
            out_ref.at[b, t, pl.ds(row0, nrows)],
            out_sems[pool].at[slot],
        )

    by_pool_idx = {(c[4], c[6]): i for i, c in enumerate(chunks)}
    next_start = [by_pool_idx.get((c[4], c[6] + _NBUF[c[4]])) for c in chunks]
    prev_out = [by_pool_idx.get((c[4], c[6] - _NBUF[c[4]])) for c in chunks]

    for k in range(len(chunks)):
        if chunks[k][6] < _NBUF[chunks[k][4]]:
            in_copy(k).start(priority=1)

    for k in range(len(chunks)):
        b, t, row0, nrows, pool, slot, _ = chunks[k]
        in_copy(k).wait()
        if prev_out[k] is not None:
            out_copy(prev_out[k]).wait()
        outbufs[pool][slot] = inbufs[pool][slot] + rows_ref[b * T + t]
        out_copy(k).start()
        if next_start[k] is not None:
            in_copy(next_start[k]).start(priority=1)

    for k in range(len(chunks)):
        if next_start[k] is None:
            out_copy(k).wait()


def kernel(hidden_state, aspect_ratio_ids, embedding_table, gate):
    B, T, P, H = hidden_state.shape
    emb = embedding_table.reshape(-1, T, 1, H)
    ids = aspect_ratio_ids.astype(jnp.int32)
    gate2d = gate.reshape(1, 1)

    grid_spec = pltpu.PrefetchScalarGridSpec(
        num_scalar_prefetch=1,
        grid=(1,),
        in_specs=[
            pl.BlockSpec(memory_space=pl.ANY),
            pl.BlockSpec((emb.shape[0], T, 1, H), lambda i, ids_ref: (0, 0, 0, 0)),
            pl.BlockSpec((1, 1), lambda i, ids_ref: (0, 0)),
        ],
        out_specs=pl.BlockSpec(memory_space=pl.ANY),
        scratch_shapes=[
            pltpu.VMEM((B * T, 1, H), jnp.float32),
            pltpu.VMEM((_NBUF[0], _CP, H), jnp.float32),
            pltpu.VMEM((_NBUF[0], _CP, H), jnp.float32),
            pltpu.VMEM((_NBUF[1], _CP_LAST, H), jnp.float32),
            pltpu.VMEM((_NBUF[1], _CP_LAST, H), jnp.float32),
            pltpu.SemaphoreType.DMA((_NBUF[0],)),
            pltpu.SemaphoreType.DMA((_NBUF[0],)),
            pltpu.SemaphoreType.DMA((_NBUF[1],)),
            pltpu.SemaphoreType.DMA((_NBUF[1],)),
        ],
    )
    return pl.pallas_call(
        _body,
        grid_spec=grid_spec,
        out_shape=jax.ShapeDtypeStruct((B, T, P, H), hidden_state.dtype),
    )(ids, hidden_state, emb, gate2d)


# speedup vs baseline: 3.6385x; 1.0003x over previous
import jax
import jax.numpy as jnp
from jax.experimental import pallas as pl
from jax.experimental.pallas import tpu as pltpu

_NBUF = (3, 1)
_CP = 1601
_CP_LAST = max(1601 - (1601 // _CP) * _CP, 1)


def _schedule(B, T, P):
    chunks = []
    counts = [0, 0]
    for b in range(B):
        for t in range(T):
            r = 0
            while r < P:
                n = min(_CP, P - r)
                pool = 0 if n == _CP else 1
                idx = counts[pool]
                counts[pool] += 1
                chunks.append((b, t, r, n, pool, idx % _NBUF[pool], idx))
                r += n
    return chunks


def _body(ids_ref, hid_ref, emb_ref, gate_ref, out_ref,
          rows_ref, in0, out0, in1, out1,
          in_sems0, out_sems0, in_sems1, out_sems1):
    B, T, P, H = hid_ref.shape
    chunks = _schedule(B, T, P)
    inbufs = (in0, in1)
    outbufs = (out0, out1)
    in_sems = (in_sems0, in_sems1)
    out_sems = (out_sems0, out_sems1)

    g = jnp.tanh(gate_ref[...])
    for seg in range(B * T):
        b, t = divmod(seg, T)
        row = ids_ref[b]
        rows_ref[seg] = emb_ref[row, t] * g

    def in_copy(k):
        b, t, row0, nrows, pool, slot, _ = chunks[k]
        return pltpu.make_async_copy(
            hid_ref.at[b, t, pl.ds(row0, nrows)],
            inbufs[pool].at[slot],
            in_sems[pool].at[slot],
        )

    def out_copy(k):
        b, t, row0, nrows, pool, slot, _ = chunks[k]
        return pltpu.make_async_copy(
            outbufs[pool].at[slot],
            out_ref.at[b, t, pl.ds(row0, nrows)],
            out_sems[pool].at[slot],
        )

    by_pool_idx = {(c[4], c[6]): i for i, c in enumerate(chunks)}
    next_start = [by_pool_idx.get((c[4], c[6] + _NBUF[c[4]])) for c in chunks]
    prev_out = [by_pool_idx.get((c[4], c[6] - _NBUF[c[4]])) for c in chunks]

    for k in range(len(chunks)):
        if chunks[k][6] < _NBUF[chunks[k][4]]:
            in_copy(k).start(priority=1)

    for k in range(len(chunks)):
        b, t, row0, nrows, pool, slot, _ = chunks[k]
        in_copy(k).wait()
        if prev_out[k] is not None:
            out_copy(prev_out[k]).wait()
        outbufs[pool][slot] = inbufs[pool][slot] + rows_ref[b * T + t]
        out_copy(k).start()
        if next_start[k] is not None:
            in_copy(next_start[k]).start(priority=1)

    for k in range(len(chunks)):
        if next_start[k] is None:
            out_copy(k).wait()


def kernel(hidden_state, aspect_ratio_ids, embedding_table, gate):
    B, T, P, H = hidden_state.shape
    emb = embedding_table.reshape(-1, T, 1, H)
    ids = aspect_ratio_ids.astype(jnp.int32)
    gate2d = gate.reshape(1, 1)

    grid_spec = pltpu.PrefetchScalarGridSpec(
        num_scalar_prefetch=1,
        grid=(1,),
        in_specs=[
            pl.BlockSpec(memory_space=pl.ANY),
            pl.BlockSpec((emb.shape[0], T, 1, H), lambda i, ids_ref: (0, 0, 0, 0)),
            pl.BlockSpec((1, 1), lambda i, ids_ref: (0, 0)),
        ],
        out_specs=pl.BlockSpec(memory_space=pl.ANY),
        scratch_shapes=[
            pltpu.VMEM((B * T, 1, H), jnp.float32),
            pltpu.VMEM((_NBUF[0], _CP, H), jnp.float32),
            pltpu.VMEM((_NBUF[0], _CP, H), jnp.float32),
            pltpu.VMEM((_NBUF[1], _CP_LAST, H), jnp.float32),
            pltpu.VMEM((_NBUF[1], _CP_LAST, H), jnp.float32),
            pltpu.SemaphoreType.DMA((_NBUF[0],)),
            pltpu.SemaphoreType.DMA((_NBUF[0],)),
            pltpu.SemaphoreType.DMA((_NBUF[1],)),
            pltpu.SemaphoreType.DMA((_NBUF[1],)),
        ],
    )
    return pl.pallas_call(
        _body,
        grid_spec=grid_spec,
        out_shape=jax.ShapeDtypeStruct((B, T, P, H), hidden_state.dtype),
    )(ids, hidden_state, emb, gate2d)
